# baseline (speedup 1.0000x reference)
import functools

import jax
import jax.numpy as jnp
from jax import lax
from jax.experimental import pallas as pl
from jax.experimental.pallas import tpu as pltpu
from jax.experimental.pallas import tpu_sc as plsc

B = 16
N = 256
C = 256
L = 16
BSC = 8
CW = C // 2
GB = CW // L
RU = 4


@functools.cache
def _build_sc():
  mesh = plsc.VectorSubcoreMesh(core_axis_name="c", subcore_axis_name="s")

  @functools.partial(
      pl.kernel,
      out_type=jax.ShapeDtypeStruct((BSC, C), jnp.int32),
      mesh=mesh,
      scratch_types=[
          pltpu.VMEM((N, CW), jnp.float32),
          pltpu.VMEM((CW,), jnp.int32),
      ],
  )
  def _argmax_sc(x_hbm, out_hbm, xbuf, obuf):
    cid = lax.axis_index("c")
    sid = lax.axis_index("s")
    b = sid // 2
    h = sid % 2

    @pl.when(cid == h)
    def _work():
      pltpu.sync_copy(x_hbm.at[b, :, pl.ds(h * CW, CW)], xbuf)

      sls = [pl.ds(g * L, L) for g in range(GB)]

      ninf = jnp.full((L,), -jnp.inf, jnp.float32)
      zero = jnp.zeros((L,), jnp.int32)

      @plsc.parallel_loop(0, N, 1, unroll=RU,
                          carry=((ninf,) * GB, (zero,) * GB))
      def scan(r, carry):
        bvs, bis = carry
        ri = jnp.zeros((L,), jnp.int32) + r
        nvs, nis = [], []
        for g in range(GB):
          v = xbuf[r, sls[g]]
          m = v > bvs[g]
          nvs.append(jnp.maximum(v, bvs[g]))
          nis.append(jnp.where(m, ri, bis[g]))
        return tuple(nvs), tuple(nis)

      _, bis = scan
      for g in range(GB):
        obuf[sls[g]] = bis[g]

      pltpu.sync_copy(obuf, out_hbm.at[b, pl.ds(h * CW, CW)])

  return _argmax_sc


def _argmax_tc_body(x_ref, o_ref):
  x2 = x_ref[0]
  m = jnp.max(x2, axis=0)
  rows = lax.broadcasted_iota(jnp.int32, (N, C), 0)
  masked = jnp.where(x2 == m[None, :], rows, N)
  o_ref[0, 0] = jnp.min(masked, axis=0)


@functools.cache
def _build_tc():
  return pl.pallas_call(
      _argmax_tc_body,
      grid=(B - BSC,),
      in_specs=[pl.BlockSpec((1, N, C), lambda i: (i + BSC, 0, 0))],
      out_specs=pl.BlockSpec((1, 1, C), lambda i: (i, 0, 0)),
      out_shape=jax.ShapeDtypeStruct((B - BSC, 1, C), jnp.int32),
  )


def kernel(x):
    idx_sc = _build_sc()(x)
    idx_tc = _build_tc()(x).reshape(B - BSC, C)
    idx = jnp.concatenate([idx_sc, idx_tc], axis=0)
    return idx.astype(jnp.int64)


# speedup vs baseline: 1.0180x; 1.0180x over previous
import functools

import jax
import jax.numpy as jnp
from jax import lax
from jax.experimental import pallas as pl
from jax.experimental.pallas import tpu as pltpu
from jax.experimental.pallas import tpu_sc as plsc

B = 16
N = 256
C = 256
L = 16
NS = 16
BSC = 8
CW = C // 2
RW = N // 2
GB = CW // L
RU = 4


@functools.cache
def _build_sc():
  mesh = plsc.VectorSubcoreMesh(core_axis_name="c", subcore_axis_name="s")

  @functools.partial(
      pl.kernel,
      out_type=jax.ShapeDtypeStruct((BSC, C), jnp.int32),
      mesh=mesh,
      scratch_types=[
          pltpu.VMEM((RW, CW), jnp.float32),
          pltpu.VMEM((CW,), jnp.float32),
          pltpu.VMEM((CW,), jnp.int32),
          pltpu.VMEM((CW,), jnp.float32),
          pltpu.VMEM((CW,), jnp.int32),
          pltpu.VMEM((CW,), jnp.int32),
          pltpu.VMEM_SHARED((NS, CW), jnp.float32),
          pltpu.VMEM_SHARED((NS, CW), jnp.int32),
      ],
  )
  def _argmax_sc(x_hbm, out_hbm, xbuf, mymax, myidx, pmax, pidx, obuf,
                 shmax, shidx):
    cid = lax.axis_index("c")
    sid = lax.axis_index("s")
    b = sid // 2
    rh = sid % 2
    r0 = rh * RW

    pltpu.sync_copy(
        x_hbm.at[b, pl.ds(r0, RW), pl.ds(cid * CW, CW)], xbuf)

    sls = [pl.ds(g * L, L) for g in range(GB)]

    ninf = jnp.full((L,), -jnp.inf, jnp.float32)
    zero = jnp.zeros((L,), jnp.int32)

    @plsc.parallel_loop(0, RW, 1, unroll=RU,
                        carry=((ninf,) * GB, (zero,) * GB))
    def scan(r, carry):
      bvs, bis = carry
      ri = jnp.zeros((L,), jnp.int32) + (r + r0)
      nvs, nis = [], []
      for g in range(GB):
        v = xbuf[r, sls[g]]
        m = v > bvs[g]
        nvs.append(jnp.maximum(v, bvs[g]))
        nis.append(jnp.where(m, ri, bis[g]))
      return tuple(nvs), tuple(nis)

    bvs, bis = scan
    for g in range(GB):
      mymax[sls[g]] = bvs[g]
      myidx[sls[g]] = bis[g]

    pltpu.sync_copy(mymax, shmax.at[sid])
    pltpu.sync_copy(myidx, shidx.at[sid])
    plsc.subcore_barrier()

    @pl.when(rh == 0)
    def _combine():
      pltpu.sync_copy(shmax.at[sid + 1], pmax)
      pltpu.sync_copy(shidx.at[sid + 1], pidx)
      for g in range(GB):
        take_hi = pmax[sls[g]] > mymax[sls[g]]
        obuf[sls[g]] = jnp.where(take_hi, pidx[sls[g]], myidx[sls[g]])
      pltpu.sync_copy(obuf, out_hbm.at[b, pl.ds(cid * CW, CW)])

  return _argmax_sc


def _argmax_tc_body(x_ref, o_ref):
  x2 = x_ref[0]
  m = jnp.max(x2, axis=0)
  rows = lax.broadcasted_iota(jnp.int32, (N, C), 0)
  masked = jnp.where(x2 == m[None, :], rows, N)
  o_ref[0, 0] = jnp.min(masked, axis=0)


@functools.cache
def _build_tc():
  return pl.pallas_call(
      _argmax_tc_body,
      grid=(B - BSC,),
      in_specs=[pl.BlockSpec((1, N, C), lambda i: (i + BSC, 0, 0))],
      out_specs=pl.BlockSpec((1, 1, C), lambda i: (i, 0, 0)),
      out_shape=jax.ShapeDtypeStruct((B - BSC, 1, C), jnp.int32),
  )


def kernel(x):
    idx_sc = _build_sc()(x)
    idx_tc = _build_tc()(x).reshape(B - BSC, C)
    idx = jnp.concatenate([idx_sc, idx_tc], axis=0)
    return idx.astype(jnp.int64)


# speedup vs baseline: 1.0189x; 1.0009x over previous
import functools

import jax
import jax.numpy as jnp
from jax import lax
from jax.experimental import pallas as pl
from jax.experimental.pallas import tpu as pltpu
from jax.experimental.pallas import tpu_sc as plsc

B = 16
N = 256
C = 256
L = 16
NS = 16
RW = N // 2
GROUPS = C // L
GB = 8
RU = 4


@functools.cache
def _build():
  mesh = plsc.VectorSubcoreMesh(core_axis_name="c", subcore_axis_name="s")

  @functools.partial(
      pl.kernel,
      out_type=jax.ShapeDtypeStruct((B, C), jnp.int32),
      mesh=mesh,
      scratch_types=[
          pltpu.VMEM((RW, C), jnp.float32),
          pltpu.VMEM((C,), jnp.float32),
          pltpu.VMEM((C,), jnp.int32),
          pltpu.VMEM((C,), jnp.float32),
          pltpu.VMEM((C,), jnp.int32),
          pltpu.VMEM((C,), jnp.int32),
          pltpu.VMEM_SHARED((NS, C), jnp.float32),
          pltpu.VMEM_SHARED((NS, C), jnp.int32),
      ],
  )
  def _argmax_sc(x_hbm, out_hbm, xbuf, mymax, myidx, pmax, pidx, obuf,
                 shmax, shidx):
    cid = lax.axis_index("c")
    sid = lax.axis_index("s")
    b = cid * (NS // 2) + sid // 2
    rh = sid % 2
    r0 = rh * RW

    pltpu.sync_copy(x_hbm.at[b, pl.ds(r0, RW)], xbuf)

    for blk in range(GROUPS // GB):
      sls = [pl.ds((blk * GB + g) * L, L) for g in range(GB)]

      ninf = jnp.full((L,), -jnp.inf, jnp.float32)
      zero = jnp.zeros((L,), jnp.int32)

      @plsc.parallel_loop(0, RW, 1, unroll=RU,
                          carry=((ninf,) * GB, (zero,) * GB))
      def scan(r, carry, sls=sls):
        bvs, bis = carry
        ri = jnp.zeros((L,), jnp.int32) + (r + r0)
        nvs, nis = [], []
        for g in range(GB):
          v = xbuf[r, sls[g]]
          m = v > bvs[g]
          nvs.append(jnp.maximum(v, bvs[g]))
          nis.append(jnp.where(m, ri, bis[g]))
        return tuple(nvs), tuple(nis)

      bvs, bis = scan
      for g in range(GB):
        mymax[sls[g]] = bvs[g]
        myidx[sls[g]] = bis[g]

    pltpu.sync_copy(mymax, shmax.at[sid])
    pltpu.sync_copy(myidx, shidx.at[sid])
    plsc.subcore_barrier()

    @pl.when(rh == 0)
    def _combine():
      pltpu.sync_copy(shmax.at[sid + 1], pmax)
      pltpu.sync_copy(shidx.at[sid + 1], pidx)
      for g in range(GROUPS):
        sl = pl.ds(g * L, L)
        take_hi = pmax[sl] > mymax[sl]
        obuf[sl] = jnp.where(take_hi, pidx[sl], myidx[sl])
      pltpu.sync_copy(obuf, out_hbm.at[b])

  return _argmax_sc


def kernel(x):
    idx = _build()(x)
    return idx.astype(jnp.int64)


# speedup vs baseline: 1.0549x; 1.0353x over previous
import functools

import jax
import jax.numpy as jnp
from jax import lax
from jax.experimental import pallas as pl
from jax.experimental.pallas import tpu as pltpu
from jax.experimental.pallas import tpu_sc as plsc

B = 16
N = 256
C = 256
L = 16
CW = C // 2
GB = 8
RU = 4


@functools.cache
def _build():
  mesh = plsc.VectorSubcoreMesh(core_axis_name="c", subcore_axis_name="s")

  @functools.partial(
      pl.kernel,
      out_type=jax.ShapeDtypeStruct((B, C), jnp.int32),
      mesh=mesh,
      scratch_types=[
          pltpu.VMEM((N, CW), jnp.float32),
          pltpu.VMEM((CW,), jnp.int32),
      ],
  )
  def _argmax_sc(x_hbm, out_hbm, xbuf, obuf):
    h = lax.axis_index("c")
    b = lax.axis_index("s")

    pltpu.sync_copy(x_hbm.at[b, :, pl.ds(h * CW, CW)], xbuf)

    sls = [pl.ds(g * L, L) for g in range(GB)]

    ninf = jnp.full((L,), -jnp.inf, jnp.float32)
    zero = jnp.zeros((L,), jnp.int32)

    @plsc.parallel_loop(0, N, 1, unroll=RU,
                        carry=((ninf,) * GB, (zero,) * GB))
    def scan(r, carry):
      bvs, bis = carry
      ri = jnp.zeros((L,), jnp.int32) + r
      nvs, nis = [], []
      for g in range(GB):
        v = xbuf[r, sls[g]]
        m = v > bvs[g]
        nvs.append(jnp.maximum(v, bvs[g]))
        nis.append(jnp.where(m, ri, bis[g]))
      return tuple(nvs), tuple(nis)

    bvs, bis = scan
    for g in range(GB):
      obuf[sls[g]] = bis[g]

    pltpu.sync_copy(obuf, out_hbm.at[b, pl.ds(h * CW, CW)])

  return _argmax_sc


def kernel(x):
    idx = _build()(x)
    return idx.astype(jnp.int64)
